# Initial kernel scaffold; baseline (speedup 1.0000x reference)
import jax
import jax.numpy as jnp
from jax.experimental import pallas as pl


def kernel(given4rank1_stimulus_set, percept_gate_weights_1, percept_gate_weights_0, E0, E1, E2, E3):
    raise NotImplementedError("write your pallas kernel here")


# speedup vs baseline: 16.6436x; 16.6436x over previous
import functools

import jax
import jax.numpy as jnp
from jax import lax
from jax.experimental import pallas as pl
from jax.experimental.pallas import tpu as pltpu
from jax.experimental.pallas import tpu_sc as plsc

_B = 16384
_NC = 2
_NS = 16
_NW = _NC * _NS
_ROWS = _B // _NW
_STEPS = _ROWS // 16

_mesh = plsc.VectorSubcoreMesh(
    core_axis_name="c", subcore_axis_name="s", num_cores=_NC, num_subcores=_NS
)


@functools.partial(
    pl.kernel,
    out_type=jax.ShapeDtypeStruct((_B * 4,), jnp.float32),
    mesh=_mesh,
    compiler_params=pltpu.CompilerParams(needs_layout_passes=False),
    scratch_types=[
        pltpu.VMEM((_ROWS * 5,), jnp.int32),
        pltpu.VMEM((_ROWS * 2,), jnp.float32),
        pltpu.VMEM((_ROWS * 2,), jnp.float32),
        pltpu.VMEM((256,), jnp.float32),
        pltpu.VMEM((_ROWS * 4,), jnp.float32),
    ],
)
def _rank_sc(stim_hbm, gw1_hbm, gw0_hbm, tab_hbm, out_hbm,
             stim_v, gw1_v, gw0_v, tab_v, out_v):
    wid = lax.axis_index("s") * _NC + lax.axis_index("c")
    base = wid * _ROWS

    pltpu.sync_copy(stim_hbm.at[pl.ds(base * 5, _ROWS * 5)], stim_v)
    pltpu.sync_copy(gw1_hbm.at[pl.ds(base * 2, _ROWS * 2)], gw1_v)
    pltpu.sync_copy(gw0_hbm.at[pl.ds(base * 2, _ROWS * 2)], gw0_v)
    pltpu.sync_copy(tab_hbm, tab_v)

    lanes = lax.iota(jnp.int32, 16)

    def step(i, carry):
        row = i * 16 + lanes
        a0 = plsc.load_gather(gw0_v, [row * 2])
        g0 = plsc.load_gather(gw1_v, [row * 2])
        a1 = 1.0 - a0
        g1 = 1.0 - g0
        c0 = a0 * g0
        c1 = a0 * g1
        c2 = a1 * g0
        c3 = a1 * g1

        r5 = row * 5
        zx = []
        zy = []
        for j in range(5):
            s = plsc.load_gather(stim_v, [r5 + j])
            b = s << 3
            vx = (c0 * plsc.load_gather(tab_v, [b])
                  + c1 * plsc.load_gather(tab_v, [b + 2])
                  + c2 * plsc.load_gather(tab_v, [b + 4])
                  + c3 * plsc.load_gather(tab_v, [b + 6]))
            vy = (c0 * plsc.load_gather(tab_v, [b + 1])
                  + c1 * plsc.load_gather(tab_v, [b + 3])
                  + c2 * plsc.load_gather(tab_v, [b + 5])
                  + c3 * plsc.load_gather(tab_v, [b + 7]))
            zx.append(vx)
            zy.append(vy)

        es = []
        for j in range(1, 5):
            dx = zx[0] - zx[j]
            dy = zy[0] - zy[j]
            q = 1.2 * dx * dx + 0.8 * dy * dy
            q = jnp.maximum(q, jnp.float32(1e-30))
            bits = lax.bitcast_convert_type(q, jnp.int32)
            bits = 0x5F3759DF - (bits >> 1)
            r = lax.bitcast_convert_type(bits, jnp.float32)
            hq = 0.5 * q
            for _ in range(3):
                r = r * (1.5 - hq * r * r)
            dist = q * r
            es.append(jnp.exp(-10.0 * dist))

        inv = 1.0 / (es[0] + es[1] + es[2] + es[3])
        r4 = row << 2
        for j in range(4):
            plsc.store_scatter(out_v, [r4 + j], es[j] * inv)
        return carry

    lax.fori_loop(0, _STEPS, step, 0)
    pltpu.sync_copy(out_v, out_hbm.at[pl.ds(base * 4, _ROWS * 4)])


def kernel(given4rank1_stimulus_set, percept_gate_weights_1,
           percept_gate_weights_0, E0, E1, E2, E3):
    stim = given4rank1_stimulus_set.astype(jnp.int32).reshape(-1)
    gw1 = percept_gate_weights_1.reshape(-1)
    gw0 = percept_gate_weights_0.reshape(-1)
    tab = jnp.concatenate(
        [E0, E1, E2, E3], axis=1)
    tab = jnp.concatenate(
        [tab, jnp.zeros((1, 8), jnp.float32)], axis=0).reshape(-1)
    out = _rank_sc(stim, gw1, gw0, tab)
    return out.reshape(_B, 4)
